# baseline (speedup 1.0000x reference)
import jax
import jax.numpy as jnp
from jax import lax
from jax.experimental import pallas as pl
from jax.experimental.pallas import tpu as pltpu
from jax.experimental.pallas import tpu_sc as plsc

N = 10000
E = 320000
HEADS = 8
DIM = 16
HD = HEADS * DIM

NC = 2
NS = 16
NW = NC * NS
CH = 32
NBLK = E // CH
NFULL = NBLK // NW
XTRA = NBLK - NFULL * NW
GB = 16
RPW = 640
NROW = NS * RPW
ZRPW = RPW // 16
NROWZ = NROW // 16

F32 = jnp.float32

_GDN = lax.GatherDimensionNumbers(
    offset_dims=(), collapsed_slice_dims=(0,), start_index_map=(0,))


def _lane_shuffle(v, perm):
    return lax.gather(v, perm, _GDN, slice_sizes=(1,),
                      mode=lax.GatherScatterMode.PROMISE_IN_BOUNDS)


def _qkv_body(h_ref, w_ref, b_ref, q_ref, k_ref, v_ref):
    o = jnp.dot(h_ref[...], w_ref[...], preferred_element_type=F32) + b_ref[...]
    q_ref[...] = o[:, 0:HD]
    k_ref[...] = o[:, HD:2 * HD]
    v_ref[...] = o[:, 2 * HD:3 * HD]


def _proj_body(e_ref, w_ref, b_ref, o_ref):
    o_ref[...] = jnp.dot(e_ref[...], w_ref[...], preferred_element_type=F32) + b_ref[...]


def _final_body(wv0_ref, wv1_ref, zx0_ref, zx1_ref, o_ref):
    z = zx0_ref[0] + zx1_ref[0] + 1e-6
    o_ref[...] = (wv0_ref[0] + wv1_ref[0]) / z


def _edge_body(qt, kt, vt, pe, e4,
               eout, wvp, zxp,
               kv0, kv1, qv0, qv1, vv0, vv1, pv0, pv1,
               ib, dsc0, dsc1, zsc0, zsc1, zrow, ov,
               wv_sh, z_sh, *sems):
    kv = [kv0, kv1]
    qv = [qv0, qv1]
    vv = [vv0, vv1]
    pv = [pv0, pv1]
    dsc = [dsc0, dsc1]
    zsc = [zsc0, zsc1]
    semg = [sems[0:4], sems[4:8]]
    sems_st = [sems[8:11], sems[11:14]]

    cid = lax.axis_index("c")
    sid = lax.axis_index("s")
    wid = cid * NS + sid
    lane = lax.broadcasted_iota(jnp.int32, (16,), 0)
    zvec = jnp.zeros((16,), F32)
    bfly = [jnp.reshape(lane ^ (1 << k), (16, 1)) for k in range(4)]

    def zero_body(i, carry):
        for c in range(HEADS):
            kv0[i, pl.ds(16 * c, 16)] = zvec
        return carry

    lax.fori_loop(0, CH, zero_body, 0)
    r0 = sid * RPW
    for j in range(RPW // CH):
        pltpu.sync_copy(kv0, wv_sh.at[pl.ds(r0 + j * CH, CH)])
    zr0 = sid * ZRPW
    for j in range(ZRPW // CH):
        pltpu.sync_copy(kv0, z_sh.at[pl.ds(zr0 + j * CH, CH)])
    rem = ZRPW - (ZRPW // CH) * CH
    if rem:
        pltpu.sync_copy(kv0.at[pl.ds(0, rem)],
                        z_sh.at[pl.ds(zr0 + (ZRPW // CH) * CH, rem)])
    plsc.subcore_barrier()

    def ebase(c):
        return jnp.minimum((c * NW + wid) * CH, E - CH)

    def regcopy(c, st):
        row = c & (GB - 1)
        d0 = ib[row, pl.ds(CH, 16)]
        d1 = ib[row, pl.ds(CH + 16, 16)]
        dsc[st][pl.ds(0, 16)] = d0
        dsc[st][pl.ds(16, 16)] = d1
        zsc[st][pl.ds(0, 16)] = lax.shift_right_logical(d0, 4)
        zsc[st][pl.ds(16, 16)] = lax.shift_right_logical(d1, 4)

    def gather_descs(c, st):
        row = c & (GB - 1)
        return [
            pltpu.make_async_copy(kt.at[ib.at[row, pl.ds(0, CH)]], kv[st],
                                  semg[st][0]),
            pltpu.make_async_copy(qt.at[dsc[st]], qv[st], semg[st][1]),
            pltpu.make_async_copy(vt.at[ib.at[row, pl.ds(0, CH)]], vv[st],
                                  semg[st][2]),
            pltpu.make_async_copy(pe.at[pl.ds(ebase(c), CH)], pv[st],
                                  semg[st][3]),
        ]

    def gather_issue(c, st):
        for d in gather_descs(c, st):
            d.start()

    def gather_wait(c, st):
        for d in gather_descs(c, st):
            d.wait()

    def compute(st):
        def group(g, carry):
            dvec = dsc[st][pl.ds(g * 16, 16)]
            mvec = dvec & 15
            for j in range(16):
                e = g * 16 + j
                srow = zvec
                for hh in range(HEADS):
                    sl = pl.ds(16 * hh, 16)
                    sc = kv[st][e, sl] * qv[st][e, sl] * pv[st][e, sl]
                    pv[st][e, sl] = sc
                    tot = sc
                    for perm in bfly:
                        tot = tot + _lane_shuffle(tot, perm)
                    es = jnp.exp(jnp.clip(tot, -5.0, 5.0))
                    vv[st][e, sl] = vv[st][e, sl] * es
                    srow = jnp.where(lane == hh, es, srow)
                m = mvec[j]
                off8 = (m & 1) * 8
                placed = _lane_shuffle(
                    srow, jnp.reshape((lane - off8) & 15, (16, 1)))
                for b in range(HEADS):
                    qv[st][e, pl.ds(16 * b, 16)] = zvec
                qv[st][e, pl.ds((m >> 1) * 16, 16)] = placed
            return carry

        lax.fori_loop(0, CH // 16, group, 0)

    def store_issue(c, st):
        pltpu.async_copy(pv[st], eout.at[pl.ds(ebase(c), CH)], sems_st[st][0])
        pltpu.async_copy(vv[st], wv_sh.at[dsc[st]], sems_st[st][1], add=True)
        pltpu.async_copy(qv[st], z_sh.at[zsc[st]], sems_st[st][2], add=True)

    def store_wait(c, st):
        pltpu.make_async_copy(pv[st], eout.at[pl.ds(ebase(c), CH)],
                              sems_st[st][0]).wait()
        pltpu.make_async_copy(vv[st], wv_sh.at[dsc[st]], sems_st[st][1]).wait()
        pltpu.make_async_copy(qv[st], z_sh.at[zsc[st]], sems_st[st][2]).wait()

    pltpu.sync_copy(e4.at[wid, pl.ds(0, GB)], ib)
    regcopy(0, 0)
    gather_issue(0, 0)

    def pair(p, carry):
        for b in range(2):
            c = 2 * p + b
            s = b
            sn = 1 - b
            cn = c + 1
            gather_wait(c, s)
            if b == 1:
                store_wait(c - 1, sn)
            else:
                @pl.when(p > 0)
                def _():
                    store_wait(c - 1, sn)

            @pl.when((cn & (GB - 1)) == 0)
            def _():
                pltpu.sync_copy(
                    e4.at[wid, pl.ds(pl.multiple_of(cn, GB), GB)], ib)

            regcopy(cn, sn)
            gather_issue(cn, sn)
            compute(s)
            store_issue(c, s)
        return carry

    lax.fori_loop(0, NFULL // 2, pair, 0)

    store_wait(NFULL - 1, 1)
    gather_wait(NFULL, 0)

    @pl.when(wid < XTRA)
    def _():
        compute(0)
        pltpu.sync_copy(pv[0], eout.at[pl.ds(ebase(NFULL), CH)])
        pltpu.sync_copy(vv[0], wv_sh.at[dsc[0]], add=True)
        pltpu.sync_copy(qv[0], z_sh.at[zsc[0]], add=True)

    plsc.subcore_barrier()

    pltpu.sync_copy(wv_sh.at[pl.ds(r0, RPW)], wvp.at[cid, pl.ds(r0, RPW)])

    def zdump(ri, carry):
        row = zr0 + ri
        pltpu.sync_copy(z_sh.at[row], zrow)

        def node_body(r, c2):
            vb = zrow[pl.ds((r >> 1) * 16, 16)]
            for hh in range(HEADS):
                p = (r & 1) * 8 + hh
                t = jnp.where(lane == p, vb, 0.0)
                for perm in bfly:
                    t = t + _lane_shuffle(t, perm)
                ov[r, pl.ds(16 * hh, 16)] = t
            return c2

        lax.fori_loop(0, 16, node_body, 0)
        pltpu.sync_copy(ov, zxp.at[cid, pl.ds(row * 16, 16)])
        return carry

    lax.fori_loop(0, ZRPW, zdump, 0)


@jax.jit
def kernel(h, e, edge_index, W_Q, b_Q, W_K, b_K, W_V, b_V, W_E, b_E):
    w_qkv = jnp.concatenate([W_Q, W_K * 0.25, W_V], axis=1)
    b_qkv = jnp.concatenate([b_Q, b_K * 0.25, b_V]).reshape(1, 3 * HD)

    qkv_call = pl.pallas_call(
        _qkv_body,
        grid=(5,),
        in_specs=[
            pl.BlockSpec((2000, HD), lambda i: (i, 0)),
            pl.BlockSpec((HD, 3 * HD), lambda i: (0, 0)),
            pl.BlockSpec((1, 3 * HD), lambda i: (0, 0)),
        ],
        out_specs=[pl.BlockSpec((2000, HD), lambda i: (i, 0))] * 3,
        out_shape=[jax.ShapeDtypeStruct((N, HD), F32)] * 3,
    )
    q_t, k_t, v_t = qkv_call(h, w_qkv, b_qkv)

    proj_call = pl.pallas_call(
        _proj_body,
        grid=(125,),
        in_specs=[
            pl.BlockSpec((2560, HD), lambda i: (i, 0)),
            pl.BlockSpec((HD, HD), lambda i: (0, 0)),
            pl.BlockSpec((1, HD), lambda i: (0, 0)),
        ],
        out_specs=pl.BlockSpec((2560, HD), lambda i: (i, 0)),
        out_shape=jax.ShapeDtypeStruct((E, HD), F32),
    )
    pe = proj_call(e, W_E, b_E.reshape(1, HD))

    e4 = edge_index.reshape(2, NBLK, CH).transpose(1, 0, 2).reshape(NBLK, 2 * CH)
    e4 = jnp.pad(e4, ((0, (NFULL + 1) * NW - NBLK), (0, 0)))
    e4 = e4.reshape(NFULL + 1, NW, 2 * CH).transpose(1, 0, 2)
    e4 = jnp.pad(e4, ((0, 0), (0, GB - 1 - (NFULL % GB)), (0, 0)))

    mesh = plsc.VectorSubcoreMesh(
        core_axis_name="c", subcore_axis_name="s", num_cores=NC, num_subcores=NS)
    edge_call = pl.kernel(
        _edge_body,
        out_type=[
            jax.ShapeDtypeStruct((E, HD), F32),
            jax.ShapeDtypeStruct((NC, NROW, HD), F32),
            jax.ShapeDtypeStruct((NC, NROW, HD), F32),
        ],
        mesh=mesh,
        scratch_types=(
            [pltpu.VMEM((CH, HD), F32)] * 8
            + [pltpu.VMEM((GB, 2 * CH), jnp.int32)]
            + [pltpu.VMEM((CH,), jnp.int32)] * 4
            + [pltpu.VMEM((HD,), F32), pltpu.VMEM((16, HD), F32)]
            + [pltpu.VMEM_SHARED((NROW, HD), F32),
               pltpu.VMEM_SHARED((NROWZ, HD), F32)]
            + [pltpu.SemaphoreType.DMA] * 14
        ),
    )
    eout, wvp, zxp = edge_call(q_t, k_t, v_t, pe, e4)

    final_call = pl.pallas_call(
        _final_body,
        grid=(5,),
        in_specs=[
            pl.BlockSpec((1, 2000, HD), lambda i: (0, i, 0)),
            pl.BlockSpec((1, 2000, HD), lambda i: (1, i, 0)),
            pl.BlockSpec((1, 2000, HD), lambda i: (0, i, 0)),
            pl.BlockSpec((1, 2000, HD), lambda i: (1, i, 0)),
        ],
        out_specs=pl.BlockSpec((2000, HD), lambda i: (i, 0)),
        out_shape=jax.ShapeDtypeStruct((N, HD), F32),
    )
    h_out = final_call(wvp, wvp, zxp, zxp)

    return (h_out.reshape(N, HEADS, DIM), eout.reshape(E, HEADS, DIM))


# speedup vs baseline: 1.0225x; 1.0225x over previous
import jax
import jax.numpy as jnp
from jax import lax
from jax.experimental import pallas as pl
from jax.experimental.pallas import tpu as pltpu
from jax.experimental.pallas import tpu_sc as plsc

N = 10000
E = 320000
HEADS = 8
DIM = 16
HD = HEADS * DIM

NC = 2
NS = 16
NW = NC * NS
CH = 32
NBLK = E // CH
NFULL = NBLK // NW
XTRA = NBLK - NFULL * NW
GB = 16
RPW = 640
NROW = NS * RPW
ZRPW = RPW // 16
NROWZ = NROW // 16

F32 = jnp.float32

_GDN = lax.GatherDimensionNumbers(
    offset_dims=(), collapsed_slice_dims=(0,), start_index_map=(0,))


def _lane_shuffle(v, perm):
    return lax.gather(v, perm, _GDN, slice_sizes=(1,),
                      mode=lax.GatherScatterMode.PROMISE_IN_BOUNDS)


def _qkv_body(h_ref, w_ref, b_ref, q_ref, k_ref, v_ref):
    o = jnp.dot(h_ref[...], w_ref[...], preferred_element_type=F32) + b_ref[...]
    q_ref[...] = o[:, 0:HD]
    k_ref[...] = o[:, HD:2 * HD]
    v_ref[...] = o[:, 2 * HD:3 * HD]


def _proj_body(e_ref, w_ref, b_ref, o_ref):
    o_ref[...] = jnp.dot(e_ref[...], w_ref[...], preferred_element_type=F32) + b_ref[...]


def _final_body(wv0_ref, wv1_ref, zx0_ref, zx1_ref, o_ref):
    z = zx0_ref[0] + zx1_ref[0] + 1e-6
    o_ref[...] = (wv0_ref[0] + wv1_ref[0]) / z


def _edge_body(qt, kt, vt, pe, e4,
               eout, wvp, zxp,
               kv0, kv1, qv0, qv1, vv0, vv1, pv0, pv1,
               ib, dsc0, dsc1, zsc0, zsc1, zrow, ov,
               wv_sh, z_sh, *sems):
    kv = [kv0, kv1]
    qv = [qv0, qv1]
    vv = [vv0, vv1]
    pv = [pv0, pv1]
    dsc = [dsc0, dsc1]
    zsc = [zsc0, zsc1]
    semg = [sems[0:4], sems[4:8]]
    sems_st = [sems[8:11], sems[11:14]]

    cid = lax.axis_index("c")
    sid = lax.axis_index("s")
    wid = cid * NS + sid
    lane = lax.broadcasted_iota(jnp.int32, (16,), 0)
    zvec = jnp.zeros((16,), F32)
    bfly = [jnp.reshape(lane ^ (1 << k), (16, 1)) for k in range(4)]

    def zero_body(i, carry):
        for c in range(HEADS):
            kv0[i, pl.ds(16 * c, 16)] = zvec
        return carry

    lax.fori_loop(0, CH, zero_body, 0)
    r0 = sid * RPW
    for j in range(RPW // CH):
        pltpu.sync_copy(kv0, wv_sh.at[pl.ds(r0 + j * CH, CH)])
    zr0 = sid * ZRPW
    for j in range(ZRPW // CH):
        pltpu.sync_copy(kv0, z_sh.at[pl.ds(zr0 + j * CH, CH)])
    rem = ZRPW - (ZRPW // CH) * CH
    if rem:
        pltpu.sync_copy(kv0.at[pl.ds(0, rem)],
                        z_sh.at[pl.ds(zr0 + (ZRPW // CH) * CH, rem)])
    plsc.subcore_barrier()

    def ebase(c):
        return jnp.minimum((c * NW + wid) * CH, E - CH)

    def regcopy(c, st):
        row = c & (GB - 1)
        d0 = ib[row, pl.ds(CH, 16)]
        d1 = ib[row, pl.ds(CH + 16, 16)]
        dsc[st][pl.ds(0, 16)] = d0
        dsc[st][pl.ds(16, 16)] = d1
        zsc[st][pl.ds(0, 16)] = lax.shift_right_logical(d0, 4)
        zsc[st][pl.ds(16, 16)] = lax.shift_right_logical(d1, 4)

    def gather_descs(c, st):
        row = c & (GB - 1)
        return [
            pltpu.make_async_copy(kt.at[ib.at[row, pl.ds(0, CH)]], kv[st],
                                  semg[st][0]),
            pltpu.make_async_copy(qt.at[dsc[st]], qv[st], semg[st][1]),
            pltpu.make_async_copy(vt.at[ib.at[row, pl.ds(0, CH)]], vv[st],
                                  semg[st][2]),
            pltpu.make_async_copy(pe.at[pl.ds(ebase(c), CH)], pv[st],
                                  semg[st][3]),
        ]

    def gather_issue(c, st):
        for d in gather_descs(c, st):
            d.start()

    def gather_wait(c, st):
        for d in gather_descs(c, st):
            d.wait()

    def compute(st):
        def group(g, carry):
            dvec = dsc[st][pl.ds(g * 16, 16)]
            mvec = dvec & 15
            for j in range(16):
                e = g * 16 + j
                srow = zvec
                for hh in range(HEADS):
                    sl = pl.ds(16 * hh, 16)
                    sc = kv[st][e, sl] * qv[st][e, sl] * pv[st][e, sl]
                    pv[st][e, sl] = sc
                    tot = sc
                    for perm in bfly:
                        tot = tot + _lane_shuffle(tot, perm)
                    es = jnp.exp(jnp.clip(tot, -5.0, 5.0))
                    vv[st][e, sl] = vv[st][e, sl] * es
                    srow = jnp.where(lane == hh, es, srow)
                m = mvec[j]
                off8 = (m & 1) * 8
                placed = _lane_shuffle(
                    srow, jnp.reshape((lane - off8) & 15, (16, 1)))
                for b in range(HEADS):
                    qv[st][e, pl.ds(16 * b, 16)] = zvec
                qv[st][e, pl.ds((m >> 1) * 16, 16)] = placed
            return carry

        lax.fori_loop(0, CH // 16, group, 0)

    def store_issue(c, st):
        pltpu.async_copy(pv[st], eout.at[pl.ds(ebase(c), CH)], sems_st[st][0])
        pltpu.async_copy(vv[st], wv_sh.at[dsc[st]], sems_st[st][1], add=True)
        pltpu.async_copy(qv[st], z_sh.at[zsc[st]], sems_st[st][2], add=True)

    def store_wait(c, st):
        pltpu.make_async_copy(pv[st], eout.at[pl.ds(ebase(c), CH)],
                              sems_st[st][0]).wait()
        pltpu.make_async_copy(vv[st], wv_sh.at[dsc[st]], sems_st[st][1]).wait()
        pltpu.make_async_copy(qv[st], z_sh.at[zsc[st]], sems_st[st][2]).wait()

    pltpu.sync_copy(e4.at[wid, pl.ds(0, GB)], ib)
    regcopy(0, 0)
    gather_issue(0, 0)

    def pair(p, carry):
        for b in range(2):
            c = 2 * p + b
            s = b
            sn = 1 - b
            cn = c + 1
            gather_wait(c, s)
            if b == 1:
                store_wait(c - 1, sn)
            else:
                @pl.when(p > 0)
                def _():
                    store_wait(c - 1, sn)

            @pl.when((cn & (GB - 1)) == 0)
            def _():
                pltpu.sync_copy(
                    e4.at[wid, pl.ds(pl.multiple_of(cn, GB), GB)], ib)

            regcopy(cn, sn)
            gather_issue(cn, sn)
            compute(s)
            store_issue(c, s)
        return carry

    lax.fori_loop(0, NFULL // 2, pair, 0)

    store_wait(NFULL - 1, 1)
    gather_wait(NFULL, 0)

    @pl.when(wid < XTRA)
    def _():
        compute(0)
        pltpu.sync_copy(pv[0], eout.at[pl.ds(ebase(NFULL), CH)])
        pltpu.sync_copy(vv[0], wv_sh.at[dsc[0]], add=True)
        pltpu.sync_copy(qv[0], z_sh.at[zsc[0]], add=True)

    plsc.subcore_barrier()

    pltpu.sync_copy(wv_sh.at[pl.ds(r0, RPW)], wvp.at[cid, pl.ds(r0, RPW)])

    def zdump(ri, carry):
        row = zr0 + ri
        pltpu.sync_copy(z_sh.at[row], zrow)

        def node_body(r, c2):
            vb = zrow[pl.ds((r >> 1) * 16, 16)]
            for hh in range(HEADS):
                p = (r & 1) * 8 + hh
                t = jnp.where(lane == p, vb, 0.0)
                for perm in bfly:
                    t = t + _lane_shuffle(t, perm)
                ov[r, pl.ds(16 * hh, 16)] = t
            return c2

        lax.fori_loop(0, 16, node_body, 0)
        pltpu.sync_copy(ov, zxp.at[cid, pl.ds(row * 16, 16)])
        return carry

    lax.fori_loop(0, ZRPW, zdump, 0)


@jax.jit
def kernel(h, e, edge_index, W_Q, b_Q, W_K, b_K, W_V, b_V, W_E, b_E):
    w_qkv = jnp.concatenate([W_Q, W_K * 0.25, W_V], axis=1)
    b_qkv = jnp.concatenate([b_Q, b_K * 0.25, b_V]).reshape(1, 3 * HD)

    qkv_call = pl.pallas_call(
        _qkv_body,
        grid=(5,),
        in_specs=[
            pl.BlockSpec((2000, HD), lambda i: (i, 0)),
            pl.BlockSpec((HD, 3 * HD), lambda i: (0, 0)),
            pl.BlockSpec((1, 3 * HD), lambda i: (0, 0)),
        ],
        out_specs=[pl.BlockSpec((2000, HD), lambda i: (i, 0))] * 3,
        out_shape=[jax.ShapeDtypeStruct((N, HD), F32)] * 3,
    )
    q_t, k_t, v_t = qkv_call(h, w_qkv, b_qkv)

    proj_call = pl.pallas_call(
        _proj_body,
        grid=(80,),
        in_specs=[
            pl.BlockSpec((4000, HD), lambda i: (i, 0)),
            pl.BlockSpec((HD, HD), lambda i: (0, 0)),
            pl.BlockSpec((1, HD), lambda i: (0, 0)),
        ],
        out_specs=pl.BlockSpec((4000, HD), lambda i: (i, 0)),
        out_shape=jax.ShapeDtypeStruct((E, HD), F32),
    )
    pe = proj_call(e, W_E, b_E.reshape(1, HD))

    e4 = edge_index.reshape(2, NBLK, CH).transpose(1, 0, 2).reshape(NBLK, 2 * CH)
    e4 = jnp.pad(e4, ((0, (NFULL + 1) * NW - NBLK), (0, 0)))
    e4 = e4.reshape(NFULL + 1, NW, 2 * CH).transpose(1, 0, 2)
    e4 = jnp.pad(e4, ((0, 0), (0, GB - 1 - (NFULL % GB)), (0, 0)))

    mesh = plsc.VectorSubcoreMesh(
        core_axis_name="c", subcore_axis_name="s", num_cores=NC, num_subcores=NS)
    edge_call = pl.kernel(
        _edge_body,
        out_type=[
            jax.ShapeDtypeStruct((E, HD), F32),
            jax.ShapeDtypeStruct((NC, NROW, HD), F32),
            jax.ShapeDtypeStruct((NC, NROW, HD), F32),
        ],
        mesh=mesh,
        scratch_types=(
            [pltpu.VMEM((CH, HD), F32)] * 8
            + [pltpu.VMEM((GB, 2 * CH), jnp.int32)]
            + [pltpu.VMEM((CH,), jnp.int32)] * 4
            + [pltpu.VMEM((HD,), F32), pltpu.VMEM((16, HD), F32)]
            + [pltpu.VMEM_SHARED((NROW, HD), F32),
               pltpu.VMEM_SHARED((NROWZ, HD), F32)]
            + [pltpu.SemaphoreType.DMA] * 14
        ),
    )
    eout, wvp, zxp = edge_call(q_t, k_t, v_t, pe, e4)

    final_call = pl.pallas_call(
        _final_body,
        grid=(5,),
        in_specs=[
            pl.BlockSpec((1, 2000, HD), lambda i: (0, i, 0)),
            pl.BlockSpec((1, 2000, HD), lambda i: (1, i, 0)),
            pl.BlockSpec((1, 2000, HD), lambda i: (0, i, 0)),
            pl.BlockSpec((1, 2000, HD), lambda i: (1, i, 0)),
        ],
        out_specs=pl.BlockSpec((2000, HD), lambda i: (i, 0)),
        out_shape=jax.ShapeDtypeStruct((N, HD), F32),
    )
    h_out = final_call(wvp, wvp, zxp, zxp)

    return (h_out.reshape(N, HEADS, DIM), eout.reshape(E, HEADS, DIM))
